# baseline (speedup 1.0000x reference)
import functools

import jax
import jax.numpy as jnp
from jax import lax
from jax.experimental import pallas as pl
from jax.experimental.pallas import tpu as pltpu
from jax.experimental.pallas import tpu_sc as plsc

D = 64
B = 16384
H = 50
NC, NS = 2, 16
NW = NC * NS
BW = B // NW
Q = 256
NQ = BW // Q

_mesh = plsc.VectorSubcoreMesh(core_axis_name="c", subcore_axis_name="s")


@functools.partial(
    pl.kernel,
    mesh=_mesh,
    out_type=jax.ShapeDtypeStruct((H, D, B), jnp.float32),
    compiler_params=pltpu.CompilerParams(
        use_tc_tiling_on_sc=True,
        needs_layout_passes=False,
        disable_bounds_checks=True,
    ),
    scratch_types=[
        [pltpu.VMEM((BW,), jnp.int32) for _ in range(2)],
        [pltpu.VMEM((Q, 128), jnp.float32) for _ in range(2)],
        [pltpu.VMEM((D, Q), jnp.float32) for _ in range(2)],
        [pltpu.SemaphoreType.DMA for _ in range(2)],
        [pltpu.SemaphoreType.DMA for _ in range(2)],
        [pltpu.SemaphoreType.DMA for _ in range(2)],
    ],
)
def _gather_kernel(idx2_hbm, tab_hbm, out_hbm,
                   i2, gbuf, obuf, isem, gsem, ssem):
    wid = lax.axis_index("s") * NC + lax.axis_index("c")
    b0 = wid * BW

    def fire_idx(ib, h):
        pltpu.async_copy(idx2_hbm.at[h, pl.ds(b0, BW)], i2[ib], isem[ib])

    def wait_idx(ib):
        pltpu.make_async_copy(idx2_hbm.at[0, pl.ds(b0, BW)], i2[ib], isem[ib]).wait()

    def fire_g(qb, ib, q):
        pltpu.async_copy(tab_hbm.at[i2[ib].at[pl.ds(q * Q, Q)]], gbuf[qb], gsem[qb])

    def wait_g(qb):
        pltpu.make_async_copy(
            tab_hbm.at[i2[0].at[pl.ds(0, Q)]], gbuf[qb], gsem[qb]
        ).wait()

    def fire_s(qb, h, q):
        pltpu.async_copy(obuf[qb], out_hbm.at[h, :, pl.ds(b0 + q * Q, Q)], ssem[qb])

    def wait_s(qb):
        pltpu.make_async_copy(
            obuf[qb], out_hbm.at[0, :, pl.ds(b0, Q)], ssem[qb]
        ).wait()

    def transpose(ib, qb, q):
        @plsc.parallel_loop(0, Q // 16)
        def per_k(k):
            rows = lax.iota(jnp.int32, 16) + k * 16

            @plsc.parallel_loop(0, D, unroll=16)
            def per_f(f):
                vals = plsc.load_gather(gbuf[qb], [rows, jnp.full((16,), f, jnp.int32)])
                obuf[qb][f, pl.ds(k * 16, 16)] = vals

    def do_h(h, ib, first_h, last_h):
        wait_idx(ib)
        if not last_h:
            fire_idx(1 - ib, h + 1)
        fire_g(0, ib, 0)
        for q in range(NQ):
            qb = q % 2
            wait_g(qb)
            if q < NQ - 1:
                fire_g(1 - qb, ib, q + 1)
            if not (first_h and q < 2):
                wait_s(qb)
            transpose(ib, qb, q)
            fire_s(qb, h, q)

    fire_idx(0, 0)
    do_h(0, 0, True, False)
    do_h(1, 1, False, False)

    def h_group(g, carry):
        do_h(2 * g, 0, False, False)
        do_h(2 * g + 1, 1, False, False)
        return carry

    lax.fori_loop(1, H // 2 - 1, h_group, 0)

    do_h(H - 2, 0, False, False)
    do_h(H - 1, 1, False, True)
    wait_s(0)
    wait_s(1)


def kernel(indices, embeddings):
    idx_t = indices.T.astype(jnp.int32)
    tab2 = jnp.pad(embeddings, ((0, 0), (0, 64)))
    out3 = _gather_kernel(idx_t, tab2)
    return jnp.transpose(out3, (2, 0, 1))


# speedup vs baseline: 1.0015x; 1.0015x over previous
import functools

import jax
import jax.numpy as jnp
from jax import lax
from jax.experimental import pallas as pl
from jax.experimental.pallas import tpu as pltpu
from jax.experimental.pallas import tpu_sc as plsc

D = 64
B = 16384
H = 50
NC, NS = 2, 16
NW = NC * NS
BW = B // NW
Q = 256
NQ = BW // Q

_mesh = plsc.VectorSubcoreMesh(core_axis_name="c", subcore_axis_name="s")


@functools.partial(
    pl.kernel,
    mesh=_mesh,
    out_type=jax.ShapeDtypeStruct((H, D, B), jnp.float32),
    compiler_params=pltpu.CompilerParams(
        use_tc_tiling_on_sc=True,
        needs_layout_passes=False,
        disable_bounds_checks=True,
    ),
    scratch_types=[
        [pltpu.VMEM((BW,), jnp.int32) for _ in range(2)],
        [pltpu.VMEM((Q, 128), jnp.float32) for _ in range(2)],
        [pltpu.VMEM((D, Q), jnp.float32) for _ in range(2)],
        [pltpu.SemaphoreType.DMA for _ in range(2)],
        [pltpu.SemaphoreType.DMA for _ in range(2)],
        [pltpu.SemaphoreType.DMA for _ in range(2)],
    ],
)
def _gather_kernel(idx2_hbm, tab_hbm, out_hbm,
                   i2, gbuf, obuf, isem, gsem, ssem):
    wid = lax.axis_index("s") * NC + lax.axis_index("c")
    b0 = wid * BW

    def fire_idx(ib, h):
        pltpu.async_copy(idx2_hbm.at[h, pl.ds(b0, BW)], i2[ib], isem[ib])

    def wait_idx(ib):
        pltpu.make_async_copy(idx2_hbm.at[0, pl.ds(b0, BW)], i2[ib], isem[ib]).wait()

    def fire_g(qb, ib, q):
        pltpu.async_copy(tab_hbm.at[i2[ib].at[pl.ds(q * Q, Q)]], gbuf[qb], gsem[qb])

    def wait_g(qb):
        pltpu.make_async_copy(
            tab_hbm.at[i2[0].at[pl.ds(0, Q)]], gbuf[qb], gsem[qb]
        ).wait()

    def fire_s(qb, h, q):
        pltpu.async_copy(obuf[qb], out_hbm.at[h, :, pl.ds(b0 + q * Q, Q)], ssem[qb])

    def wait_s(qb):
        pltpu.make_async_copy(
            obuf[qb], out_hbm.at[0, :, pl.ds(b0, Q)], ssem[qb]
        ).wait()

    def transpose(ib, qb, q):
        def per_k(k, c2):
            rows = lax.iota(jnp.int32, 16) + k * 16

            @plsc.parallel_loop(0, D, unroll=8)
            def per_f(f):
                vals = plsc.load_gather(gbuf[qb], [rows, jnp.full((16,), f, jnp.int32)])
                obuf[qb][f, pl.ds(k * 16, 16)] = vals

            return c2

        lax.fori_loop(0, Q // 16, per_k, 0)

    def do_h(h, ib, first_h, last_h):
        wait_idx(ib)
        if not last_h:
            fire_idx(1 - ib, h + 1)
        fire_g(0, ib, 0)
        for q in range(NQ):
            qb = q % 2
            wait_g(qb)
            if q < NQ - 1:
                fire_g(1 - qb, ib, q + 1)
            if not (first_h and q < 2):
                wait_s(qb)
            transpose(ib, qb, q)
            fire_s(qb, h, q)

    fire_idx(0, 0)
    do_h(0, 0, True, False)
    do_h(1, 1, False, False)

    def h_group(g, carry):
        do_h(2 * g, 0, False, False)
        do_h(2 * g + 1, 1, False, False)
        return carry

    lax.fori_loop(1, H // 2 - 1, h_group, 0)

    do_h(H - 2, 0, False, False)
    do_h(H - 1, 1, False, True)
    wait_s(0)
    wait_s(1)


def kernel(indices, embeddings):
    idx_t = indices.T.astype(jnp.int32)
    tab2 = jnp.pad(embeddings, ((0, 0), (0, 64)))
    out3 = _gather_kernel(idx_t, tab2)
    return jnp.transpose(out3, (2, 0, 1))


# speedup vs baseline: 1.0358x; 1.0343x over previous
import functools

import jax
import jax.numpy as jnp
from jax import lax
from jax.experimental import pallas as pl
from jax.experimental.pallas import tpu as pltpu
from jax.experimental.pallas import tpu_sc as plsc

D = 64
B = 16384
H = 50
NC, NS = 2, 16
NW = NC * NS
BW = B // NW
Q = 128
NQ = BW // Q

_mesh = plsc.VectorSubcoreMesh(core_axis_name="c", subcore_axis_name="s")


@functools.partial(
    pl.kernel,
    mesh=_mesh,
    out_type=jax.ShapeDtypeStruct((H, D, B), jnp.float32),
    compiler_params=pltpu.CompilerParams(
        use_tc_tiling_on_sc=True,
        needs_layout_passes=False,
        disable_bounds_checks=True,
    ),
    scratch_types=[
        [pltpu.VMEM((BW,), jnp.int32) for _ in range(2)],
        [pltpu.VMEM((Q, 128), jnp.float32) for _ in range(2)],
        [pltpu.VMEM((D, Q), jnp.float32) for _ in range(2)],
        [pltpu.SemaphoreType.DMA for _ in range(2)],
        [pltpu.SemaphoreType.DMA for _ in range(2)],
        [pltpu.SemaphoreType.DMA for _ in range(2)],
    ],
)
def _gather_kernel(idx2_hbm, tab_hbm, out_hbm,
                   i2, gbuf, obuf, isem, gsem, ssem):
    wid = lax.axis_index("s") * NC + lax.axis_index("c")
    b0 = wid * BW

    def fire_idx(ib, h):
        pltpu.async_copy(idx2_hbm.at[h, pl.ds(b0, BW)], i2[ib], isem[ib])

    def wait_idx(ib):
        pltpu.make_async_copy(idx2_hbm.at[0, pl.ds(b0, BW)], i2[ib], isem[ib]).wait()

    def fire_g(qb, ib, q):
        pltpu.async_copy(tab_hbm.at[i2[ib].at[pl.ds(q * Q, Q)]], gbuf[qb], gsem[qb])

    def wait_g(qb):
        pltpu.make_async_copy(
            tab_hbm.at[i2[0].at[pl.ds(0, Q)]], gbuf[qb], gsem[qb]
        ).wait()

    def fire_s(qb, h, q):
        pltpu.async_copy(obuf[qb], out_hbm.at[h, :, pl.ds(b0 + q * Q, Q)], ssem[qb])

    def wait_s(qb):
        pltpu.make_async_copy(
            obuf[qb], out_hbm.at[0, :, pl.ds(b0, Q)], ssem[qb]
        ).wait()

    def transpose(ib, qb, q):
        def per_k(k, c2):
            rows = lax.iota(jnp.int32, 16) + k * 16

            @plsc.parallel_loop(0, D, unroll=16)
            def per_f(f):
                vals = plsc.load_gather(gbuf[qb], [rows, jnp.full((16,), f, jnp.int32)])
                obuf[qb][f, pl.ds(k * 16, 16)] = vals

            return c2

        lax.fori_loop(0, Q // 16, per_k, 0)

    def do_h(h, ib, first_h, last_h):
        wait_idx(ib)
        if not last_h:
            fire_idx(1 - ib, h + 1)
        fire_g(0, ib, 0)
        for q in range(NQ):
            qb = q % 2
            wait_g(qb)
            if q < NQ - 1:
                fire_g(1 - qb, ib, q + 1)
            if not (first_h and q < 2):
                wait_s(qb)
            transpose(ib, qb, q)
            fire_s(qb, h, q)

    fire_idx(0, 0)
    do_h(0, 0, True, False)
    do_h(1, 1, False, False)

    def h_group(g, carry):
        do_h(2 * g, 0, False, False)
        do_h(2 * g + 1, 1, False, False)
        return carry

    lax.fori_loop(1, H // 2 - 1, h_group, 0)

    do_h(H - 2, 0, False, False)
    do_h(H - 1, 1, False, True)
    wait_s(0)
    wait_s(1)


def kernel(indices, embeddings):
    idx_t = indices.T.astype(jnp.int32)
    tab2 = jnp.pad(embeddings, ((0, 0), (0, 64)))
    out3 = _gather_kernel(idx_t, tab2)
    return jnp.transpose(out3, (2, 0, 1))


# speedup vs baseline: 1.1004x; 1.0623x over previous
import functools

import jax
import jax.numpy as jnp
from jax import lax
from jax.experimental import pallas as pl
from jax.experimental.pallas import tpu as pltpu
from jax.experimental.pallas import tpu_sc as plsc

D = 64
B = 16384
H = 50
NC, NS = 2, 16
NW = NC * NS
BW = B // NW
Q = 128
NQ = BW // Q

_mesh = plsc.VectorSubcoreMesh(core_axis_name="c", subcore_axis_name="s")


@functools.partial(
    pl.kernel,
    mesh=_mesh,
    out_type=jax.ShapeDtypeStruct((H, D, B), jnp.float32),
    compiler_params=pltpu.CompilerParams(
        use_tc_tiling_on_sc=True,
        needs_layout_passes=False,
        disable_bounds_checks=True,
    ),
    scratch_types=[
        [pltpu.VMEM((BW,), jnp.int32) for _ in range(2)],
        [pltpu.VMEM((Q, 128), jnp.float32) for _ in range(NQ)],
        [pltpu.VMEM((D, Q), jnp.float32) for _ in range(2)],
        [pltpu.SemaphoreType.DMA for _ in range(2)],
        [pltpu.SemaphoreType.DMA for _ in range(NQ)],
        [pltpu.SemaphoreType.DMA for _ in range(2)],
    ],
)
def _gather_kernel(idx_hbm, tab_hbm, out_hbm,
                   i2, gbuf, obuf, isem, gsem, ssem):
    wid = lax.axis_index("s") * NC + lax.axis_index("c")
    b0 = wid * BW

    def fire_idx(ib, h):
        pltpu.async_copy(idx_hbm.at[h, pl.ds(b0, BW)], i2[ib], isem[ib])

    def wait_idx(ib):
        pltpu.make_async_copy(idx_hbm.at[0, pl.ds(b0, BW)], i2[ib], isem[ib]).wait()

    def fire_g(gb, ib, q):
        pltpu.async_copy(tab_hbm.at[i2[ib].at[pl.ds(q * Q, Q)]], gbuf[gb], gsem[gb])

    def wait_g(gb):
        pltpu.make_async_copy(
            tab_hbm.at[i2[0].at[pl.ds(0, Q)]], gbuf[gb], gsem[gb]
        ).wait()

    def fire_s(ob, h, q):
        pltpu.async_copy(obuf[ob], out_hbm.at[h, :, pl.ds(b0 + q * Q, Q)], ssem[ob])

    def wait_s(ob):
        pltpu.make_async_copy(
            obuf[ob], out_hbm.at[0, :, pl.ds(b0, Q)], ssem[ob]
        ).wait()

    def transpose(gb, ob, q):
        def per_k(k, c2):
            rows = lax.iota(jnp.int32, 16) + k * 16

            @plsc.parallel_loop(0, D, unroll=8)
            def per_f(f):
                vals = plsc.load_gather(gbuf[gb], [rows, jnp.full((16,), f, jnp.int32)])
                obuf[ob][f, pl.ds(k * 16, 16)] = vals

            return c2

        lax.fori_loop(0, Q // 16, per_k, 0)

    def do_h(h, ib, first_h, last_h):
        if not last_h:
            fire_idx(1 - ib, h + 1)
        for q in range(NQ):
            wait_g(q)
            if q < NQ - 2:
                fire_g(q + 2, ib, q + 2)
            elif not last_h:
                if q == NQ - 2:
                    wait_idx(1 - ib)
                fire_g(q + 2 - NQ, 1 - ib, q + 2 - NQ)
            ob = q % 2
            if not (first_h and q < 2):
                wait_s(ob)
            transpose(q, ob, q)
            fire_s(ob, h, q)

    fire_idx(0, 0)
    wait_idx(0)
    fire_g(0, 0, 0)
    fire_g(1, 0, 1)
    do_h(0, 0, True, False)

    def h_group(g, carry):
        do_h(2 * g + 1, 1, False, False)
        do_h(2 * g + 2, 0, False, False)
        return carry

    lax.fori_loop(0, (H - 2) // 2, h_group, 0)

    do_h(H - 1, 1, False, True)
    wait_s(0)
    wait_s(1)


def kernel(indices, embeddings):
    idx_t = indices.T.astype(jnp.int32)
    tab2 = jnp.pad(embeddings, ((0, 0), (0, 64)))
    out3 = _gather_kernel(idx_t, tab2)
    return jnp.transpose(out3, (2, 0, 1))


# speedup vs baseline: 1.1009x; 1.0005x over previous
import functools

import jax
import jax.numpy as jnp
from jax import lax
from jax.experimental import pallas as pl
from jax.experimental.pallas import tpu as pltpu
from jax.experimental.pallas import tpu_sc as plsc

D = 64
B = 16384
H = 50
NC, NS = 2, 16
NW = NC * NS
BW = B // NW
Q = 128
NQ = BW // Q

_mesh = plsc.VectorSubcoreMesh(core_axis_name="c", subcore_axis_name="s")


@functools.partial(
    pl.kernel,
    mesh=_mesh,
    out_type=jax.ShapeDtypeStruct((H, D, B), jnp.float32),
    compiler_params=pltpu.CompilerParams(
        use_tc_tiling_on_sc=True,
        needs_layout_passes=False,
        disable_bounds_checks=True,
    ),
    scratch_types=[
        [pltpu.VMEM((BW,), jnp.int32) for _ in range(2)],
        [pltpu.VMEM((Q, 128), jnp.float32) for _ in range(NQ)],
        [pltpu.VMEM((D, Q), jnp.float32) for _ in range(2)],
        [pltpu.SemaphoreType.DMA for _ in range(2)],
        [pltpu.SemaphoreType.DMA for _ in range(NQ)],
        [pltpu.SemaphoreType.DMA for _ in range(2)],
    ],
)
def _gather_kernel(idx_hbm, tab_hbm, out_hbm,
                   i2, gbuf, obuf, isem, gsem, ssem):
    wid = lax.axis_index("s") * NC + lax.axis_index("c")
    b0 = wid * BW

    def fire_idx(ib, h):
        pltpu.async_copy(idx_hbm.at[h, pl.ds(b0, BW)], i2[ib], isem[ib])

    def wait_idx(ib):
        pltpu.make_async_copy(idx_hbm.at[0, pl.ds(b0, BW)], i2[ib], isem[ib]).wait()

    def fire_g(gb, ib, q):
        pltpu.async_copy(tab_hbm.at[i2[ib].at[pl.ds(q * Q, Q)]], gbuf[gb], gsem[gb])

    def wait_g(gb):
        pltpu.make_async_copy(
            tab_hbm.at[i2[0].at[pl.ds(0, Q)]], gbuf[gb], gsem[gb]
        ).wait()

    def fire_s(ob, h, q):
        pltpu.async_copy(obuf[ob], out_hbm.at[h, :, pl.ds(b0 + q * Q, Q)], ssem[ob])

    def wait_s(ob):
        pltpu.make_async_copy(
            obuf[ob], out_hbm.at[0, :, pl.ds(b0, Q)], ssem[ob]
        ).wait()

    def transpose(gb, ob, q):
        @plsc.parallel_loop(0, Q // 16)
        def per_k(k):
            rows = lax.iota(jnp.int32, 16) + k * 16

            @plsc.parallel_loop(0, D, unroll=8)
            def per_f(f):
                vals = plsc.load_gather(gbuf[gb], [rows, jnp.full((16,), f, jnp.int32)])
                obuf[ob][f, pl.ds(k * 16, 16)] = vals

    def do_h(h, ib, first_h, last_h):
        if not last_h:
            fire_idx(1 - ib, h + 1)
        for q in range(NQ):
            wait_g(q)
            if q < NQ - 2:
                fire_g(q + 2, ib, q + 2)
            elif not last_h:
                if q == NQ - 2:
                    wait_idx(1 - ib)
                fire_g(q + 2 - NQ, 1 - ib, q + 2 - NQ)
            ob = q % 2
            if not (first_h and q < 2):
                wait_s(ob)
            transpose(q, ob, q)
            fire_s(ob, h, q)

    fire_idx(0, 0)
    wait_idx(0)
    fire_g(0, 0, 0)
    fire_g(1, 0, 1)
    do_h(0, 0, True, False)

    def h_group(g, carry):
        do_h(2 * g + 1, 1, False, False)
        do_h(2 * g + 2, 0, False, False)
        return carry

    lax.fori_loop(0, (H - 2) // 2, h_group, 0)

    do_h(H - 1, 1, False, True)
    wait_s(0)
    wait_s(1)


def kernel(indices, embeddings):
    idx_t = indices.T.astype(jnp.int32)
    tab2 = jnp.pad(embeddings, ((0, 0), (0, 64)))
    out3 = _gather_kernel(idx_t, tab2)
    return jnp.transpose(out3, (2, 0, 1))
